# baseline (speedup 1.0000x reference)
import functools

import jax
import jax.numpy as jnp
from jax import lax
from jax.experimental import pallas as pl
from jax.experimental.pallas import tpu as pltpu
from jax.experimental.pallas import tpu_sc as plsc

N = 10000
D = 128
E = 320000

NCORES = 2
NSUB = 16
NTILES = NCORES * NSUB
CHUNK = 64
CPT = 160
BLK = 16
NBLK = CPT // BLK
EPAD = NTILES * CPT * CHUNK
ROWS_PT = 640
NPAD = NSUB * ROWS_PT
DUMMY = N
ROWS1 = 640
NPAD1 = NSUB * ROWS1
BN = 1000


def _sc_mesh():
    return plsc.VectorSubcoreMesh(core_axis_name="c", subcore_axis_name="s")


def _sc_spmv(x, src3, dst3, zeros_rows):
    dt = x.dtype

    @functools.partial(
        pl.kernel,
        mesh=_sc_mesh(),
        out_type=jax.ShapeDtypeStruct((NCORES, NPAD, D), dt),
        scratch_types=[
            pltpu.VMEM((2, BLK, CHUNK), jnp.int32),
            pltpu.VMEM((2, BLK, CHUNK), jnp.int32),
            pltpu.VMEM((4, CHUNK, D), dt),
            pltpu.VMEM_SHARED((NPAD, D), dt),
            pltpu.SemaphoreType.DMA,
            pltpu.SemaphoreType.DMA,
            pltpu.SemaphoreType.DMA,
            pltpu.SemaphoreType.DMA,
            pltpu.SemaphoreType.DMA,
            pltpu.SemaphoreType.DMA,
            pltpu.SemaphoreType.DMA,
            pltpu.SemaphoreType.DMA,
            pltpu.SemaphoreType.DMA,
            pltpu.SemaphoreType.DMA,
            pltpu.SemaphoreType.DMA,
            pltpu.SemaphoreType.DMA,
        ],
    )
    def k(x_hbm, src_hbm, dst_hbm, zeros_hbm, out_hbm,
          src_t, dst_t, rows, acc, semr0, semr1, semr2, semr3,
          semw0, semw1, semw2, semw3, sems0, sems1, semd0, semd1):
        c = lax.axis_index("c")
        s = lax.axis_index("s")
        wid = c * NSUB + s
        semr = (semr0, semr1, semr2, semr3)
        semw = (semw0, semw1, semw2, semw3)
        sems = (sems0, sems1)
        semd = (semd0, semd1)

        pltpu.sync_copy(zeros_hbm, acc.at[pl.ds(s * ROWS_PT, ROWS_PT)])
        pltpu.sync_copy(src_hbm.at[wid, pl.ds(0, BLK)], src_t.at[0])
        pltpu.sync_copy(dst_hbm.at[wid, pl.ds(0, BLK)], dst_t.at[0])
        plsc.subcore_barrier()

        def start_g(p, j, buf):
            pltpu.async_copy(x_hbm.at[src_t.at[p, j]], rows.at[buf],
                             semr[buf])

        def wait_g(buf):
            pltpu.make_async_copy(
                x_hbm.at[src_t.at[0, 0]], rows.at[buf], semr[buf]).wait()

        def start_s(p, j, buf):
            pltpu.async_copy(rows.at[buf], acc.at[dst_t.at[p, j]],
                             semw[buf], add=True)

        def wait_s(buf):
            pltpu.make_async_copy(
                rows.at[buf], acc.at[dst_t.at[0, 0]], semw[buf]).wait()

        start_g(0, 0, 0)
        start_g(0, 1, 1)

        def step(p, l, b, cross=False, first=False, last=False):
            wait_g(b)
            start_s(p, l, b)
            if not first:
                wait_s((b + 2) % 4)
            if not last:
                if cross:
                    start_g(1 - p, l + 2 - BLK, (b + 2) % 4)
                else:
                    start_g(p, l + 2, (b + 2) % 4)

        for blk in range(NBLK):
            p = blk % 2
            if blk + 1 < NBLK:
                nb = (blk + 1) * BLK
                pltpu.async_copy(src_hbm.at[wid, pl.ds(nb, BLK)],
                                 src_t.at[1 - p], sems[1 - p])
                pltpu.async_copy(dst_hbm.at[wid, pl.ds(nb, BLK)],
                                 dst_t.at[1 - p], semd[1 - p])
            if blk > 0:
                pltpu.make_async_copy(dst_hbm.at[wid, pl.ds(0, BLK)],
                                      dst_t.at[p], semd[p]).wait()

            step(p, 0, 0, first=(blk == 0))
            step(p, 1, 1, first=(blk == 0))

            def body(i, carry, p=p):
                l = 4 * i + 2
                for u in range(4):
                    step(p, l + u, (2 + u) % 4)
                return carry

            lax.fori_loop(0, (BLK - 4) // 4, body, 0)

            if blk + 1 < NBLK:
                pltpu.make_async_copy(src_hbm.at[wid, pl.ds(0, BLK)],
                                      src_t.at[1 - p], sems[1 - p]).wait()
                step(p, BLK - 2, (BLK - 2) % 4, cross=True)
                step(p, BLK - 1, (BLK - 1) % 4, cross=True)
            else:
                step(p, BLK - 2, (BLK - 2) % 4, last=True)
                step(p, BLK - 1, (BLK - 1) % 4, last=True)

        wait_s((CPT - 2) % 4)
        wait_s((CPT - 1) % 4)
        plsc.subcore_barrier()
        pltpu.sync_copy(acc.at[pl.ds(s * ROWS_PT, ROWS_PT)],
                        out_hbm.at[c].at[pl.ds(s * ROWS_PT, ROWS_PT)])

    return k(x, src3, dst3, zeros_rows)


def _sc_degrees(src3, dst3, zeros_1d, ones_1d):
    @functools.partial(
        pl.kernel,
        mesh=_sc_mesh(),
        out_type=(jax.ShapeDtypeStruct((NCORES, NPAD1), jnp.float32),
                  jax.ShapeDtypeStruct((NCORES, NPAD1), jnp.float32)),
        scratch_types=[
            pltpu.VMEM((CPT, CHUNK), jnp.int32),
            pltpu.VMEM((CPT, CHUNK), jnp.int32),
            pltpu.VMEM((CHUNK,), jnp.float32),
            pltpu.VMEM_SHARED((NPAD1,), jnp.float32),
            pltpu.VMEM_SHARED((NPAD1,), jnp.float32),
            pltpu.SemaphoreType.DMA,
            pltpu.SemaphoreType.DMA,
        ],
    )
    def k(src_hbm, dst_hbm, zeros_hbm, ones_hbm, din_hbm, dout_hbm,
          src_t, dst_t, ones_t, acc_in, acc_out, semi, semo):
        c = lax.axis_index("c")
        s = lax.axis_index("s")
        wid = c * NSUB + s
        pltpu.sync_copy(src_hbm.at[wid], src_t)
        pltpu.sync_copy(dst_hbm.at[wid], dst_t)
        pltpu.sync_copy(ones_hbm, ones_t)
        sl = pl.ds(s * ROWS1, ROWS1)
        pltpu.sync_copy(zeros_hbm, acc_in.at[sl])
        pltpu.sync_copy(zeros_hbm, acc_out.at[sl])
        plsc.subcore_barrier()

        def fire(j, carry):
            pltpu.async_copy(ones_t, acc_in.at[dst_t.at[j]], semi, add=True)
            pltpu.async_copy(ones_t, acc_out.at[src_t.at[j]], semo, add=True)
            return carry

        lax.fori_loop(0, CPT, fire, 0)

        def drain(j, carry):
            pltpu.make_async_copy(ones_t, acc_in.at[dst_t.at[0]],
                                  semi).wait()
            pltpu.make_async_copy(ones_t, acc_out.at[src_t.at[0]],
                                  semo).wait()
            return carry

        lax.fori_loop(0, CPT, drain, 0)

        plsc.subcore_barrier()
        pltpu.sync_copy(acc_in.at[sl], din_hbm.at[c].at[sl])
        pltpu.sync_copy(acc_out.at[sl], dout_hbm.at[c].at[sl])

    return k(src3, dst3, zeros_1d, ones_1d)


def _norms(dia, dib, doa, dob):
    inorm = lax.rsqrt(jnp.maximum(dia[...] + dib[...], 1.0)[:, :1])
    onorm = lax.rsqrt(jnp.maximum(doa[...] + dob[...], 1.0)[:, :1])
    return inorm, onorm


_row_spec = pl.BlockSpec((BN, D), lambda i: (i, 0))
_deg_spec = pl.BlockSpec((BN, 1), lambda i: (i, 0))
_w_spec = pl.BlockSpec((D, D), lambda i: (0, 0))
_b_spec = pl.BlockSpec((1, D), lambda i: (0, 0))


def _tc_prep(f, dia, dib, doa, dob):
    def body(f_ref, dia_r, dib_r, doa_r, dob_r, g_ref, r_ref):
        inorm, onorm = _norms(dia_r, dib_r, doa_r, dob_r)
        x = f_ref[...]
        g_ref[...] = x * onorm
        r_ref[...] = x * inorm

    return pl.pallas_call(
        body,
        grid=(N // BN,),
        in_specs=[_row_spec, _deg_spec, _deg_spec, _deg_spec, _deg_spec],
        out_specs=(_row_spec, _row_spec),
        out_shape=(jax.ShapeDtypeStruct((N, D), jnp.float32),
                   jax.ShapeDtypeStruct((N, D), jnp.float32)),
    )(f, dia, dib, doa, dob)


_p0_spec = pl.BlockSpec((1, BN, D), lambda i: (0, i, 0))
_p1_spec = pl.BlockSpec((1, BN, D), lambda i: (1, i, 0))


def _tc_layer(P, dia, dib, doa, dob, W, b, Q):
    has_res = Q is not None

    def body(p0_r, p1_r, dia_r, dib_r, doa_r, dob_r, w_r, b_r, *rest):
        g_ref = rest[-1]
        inorm, onorm = _norms(dia_r, dib_r, doa_r, dob_r)
        agg = jnp.dot(p0_r[0] + p1_r[0], w_r[...],
                      preferred_element_type=jnp.float32)
        h = agg * inorm + b_r[...]
        if has_res:
            h = h + (rest[0][0] + rest[1][0]) * inorm
        h = jnp.maximum(h, 0.0)
        g_ref[...] = h * onorm

    in_specs = [_p0_spec, _p1_spec, _deg_spec, _deg_spec, _deg_spec,
                _deg_spec, _w_spec, _b_spec]
    args = [P, P, dia, dib, doa, dob, W, b.reshape(1, D)]
    if has_res:
        in_specs += [_p0_spec, _p1_spec]
        args += [Q, Q]

    return pl.pallas_call(
        body,
        grid=(N // BN,),
        in_specs=in_specs,
        out_specs=_row_spec,
        out_shape=jax.ShapeDtypeStruct((N, D), jnp.float32),
    )(*args)


def _tc_final(P, dia, dib, W, b):
    def body(p0_r, p1_r, dia_r, dib_r, w_r, b_r, o_ref):
        inorm = lax.rsqrt(jnp.maximum(dia_r[...] + dib_r[...], 1.0)[:, :1])
        agg = jnp.dot(p0_r[0] + p1_r[0], w_r[...],
                      preferred_element_type=jnp.float32)
        o_ref[...] = agg * inorm + b_r[...]

    return pl.pallas_call(
        body,
        grid=(N // BN,),
        in_specs=[_p0_spec, _p1_spec, _deg_spec, _deg_spec, _w_spec,
                  _b_spec],
        out_specs=_row_spec,
        out_shape=jax.ShapeDtypeStruct((N, D), jnp.float32),
    )(P, P, dia, dib, W, b.reshape(1, D))


def kernel(features, edge_index, W0, b0, W1, b1, W2, b2, W3, b3):
    src = edge_index[0].astype(jnp.int32)
    dst = edge_index[1].astype(jnp.int32)
    pad = EPAD - E

    def _pad3(v, fill):
        return jnp.concatenate([v, jnp.full((pad,), fill, jnp.int32)]
                               ).reshape(NTILES, CPT, CHUNK)

    src3 = _pad3(src, 0)
    dst3 = _pad3(dst, DUMMY)
    src3_s = _pad3(src, DUMMY)
    zeros_rows = jnp.zeros((ROWS_PT, D), jnp.float32)
    zeros_1d = jnp.zeros((ROWS1,), jnp.float32)
    ones_1d = jnp.ones((CHUNK,), jnp.float32)

    din_p, dout_p = _sc_degrees(src3_s, dst3, zeros_1d, ones_1d)
    dia, dib = din_p[0, :N, None], din_p[1, :N, None]
    doa, dob = dout_p[0, :N, None], dout_p[1, :N, None]

    g, r = _tc_prep(features, dia, dib, doa, dob)
    P = _sc_spmv(g, src3, dst3, zeros_rows)
    Q = _sc_spmv(r, src3, dst3, zeros_rows)

    g1 = _tc_layer(P, dia, dib, doa, dob, W0, b0, None)
    R1 = _sc_spmv(g1, src3, dst3, zeros_rows)
    g2 = _tc_layer(R1, dia, dib, doa, dob, W1, b1, Q)
    R2 = _sc_spmv(g2, src3, dst3, zeros_rows)
    g3 = _tc_layer(R2, dia, dib, doa, dob, W2, b2, Q)
    R3 = _sc_spmv(g3, src3, dst3, zeros_rows)
    return _tc_final(R3, dia, dib, W3, b3)


# speedup vs baseline: 1.1571x; 1.1571x over previous
import functools

import jax
import jax.numpy as jnp
from jax import lax
from jax.experimental import pallas as pl
from jax.experimental.pallas import tpu as pltpu
from jax.experimental.pallas import tpu_sc as plsc

N = 10000
D = 128
E = 320000

NCORES = 2
NSUB = 16
NTILES = NCORES * NSUB
CHUNK = 128
CPT = 80
BLK = 16
NBLK = CPT // BLK
EPAD = NTILES * CPT * CHUNK
ROWS_PT = 640
NPAD = NSUB * ROWS_PT
DUMMY = N
ROWS1 = 640
NPAD1 = NSUB * ROWS1
BN = 1000


def _sc_mesh():
    return plsc.VectorSubcoreMesh(core_axis_name="c", subcore_axis_name="s")


def _sc_spmv(x, src3, dst3, zeros_rows):
    dt = x.dtype

    @functools.partial(
        pl.kernel,
        mesh=_sc_mesh(),
        out_type=jax.ShapeDtypeStruct((NCORES, NPAD, D), dt),
        scratch_types=[
            pltpu.VMEM((2, BLK, CHUNK), jnp.int32),
            pltpu.VMEM((2, BLK, CHUNK), jnp.int32),
            pltpu.VMEM((2, CHUNK, D), dt),
            pltpu.VMEM_SHARED((NPAD, D), dt),
            pltpu.SemaphoreType.DMA,
            pltpu.SemaphoreType.DMA,
            pltpu.SemaphoreType.DMA,
            pltpu.SemaphoreType.DMA,
            pltpu.SemaphoreType.DMA,
            pltpu.SemaphoreType.DMA,
            pltpu.SemaphoreType.DMA,
            pltpu.SemaphoreType.DMA,
        ],
    )
    def k(x_hbm, src_hbm, dst_hbm, zeros_hbm, out_hbm,
          src_t, dst_t, rows, acc, semr0, semr1, semw0, semw1,
          sems0, sems1, semd0, semd1):
        c = lax.axis_index("c")
        s = lax.axis_index("s")
        wid = c * NSUB + s
        semr = (semr0, semr1)
        semw = (semw0, semw1)
        sems = (sems0, sems1)
        semd = (semd0, semd1)

        pltpu.sync_copy(zeros_hbm, acc.at[pl.ds(s * ROWS_PT, ROWS_PT)])
        pltpu.sync_copy(src_hbm.at[wid, pl.ds(0, BLK)], src_t.at[0])
        pltpu.sync_copy(dst_hbm.at[wid, pl.ds(0, BLK)], dst_t.at[0])
        plsc.subcore_barrier()

        def start_g(p, j, buf):
            pltpu.async_copy(x_hbm.at[src_t.at[p, j]], rows.at[buf],
                             semr[buf])

        def wait_g(buf):
            pltpu.make_async_copy(
                x_hbm.at[src_t.at[0, 0]], rows.at[buf], semr[buf]).wait()

        def start_s(p, j, buf):
            pltpu.async_copy(rows.at[buf], acc.at[dst_t.at[p, j]],
                             semw[buf], add=True)

        def wait_s(buf):
            pltpu.make_async_copy(
                rows.at[buf], acc.at[dst_t.at[0, 0]], semw[buf]).wait()

        start_g(0, 0, 0)
        start_g(0, 1, 1)

        for blk in range(NBLK):
            p = blk % 2
            if blk + 1 < NBLK:
                nb = (blk + 1) * BLK
                pltpu.async_copy(src_hbm.at[wid, pl.ds(nb, BLK)],
                                 src_t.at[1 - p], sems[1 - p])
                pltpu.async_copy(dst_hbm.at[wid, pl.ds(nb, BLK)],
                                 dst_t.at[1 - p], semd[1 - p])
            if blk > 0:
                pltpu.make_async_copy(dst_hbm.at[wid, pl.ds(0, BLK)],
                                      dst_t.at[p], semd[p]).wait()

            def body(i, carry, p=p):
                j = 2 * i
                wait_g(0)
                start_s(p, j, 0)
                wait_g(1)
                start_s(p, j + 1, 1)
                wait_s(0)
                start_g(p, j + 2, 0)
                wait_s(1)
                start_g(p, j + 3, 1)
                return carry

            lax.fori_loop(0, BLK // 2 - 1, body, 0)

            j = BLK - 2
            wait_g(0)
            start_s(p, j, 0)
            wait_g(1)
            start_s(p, j + 1, 1)
            if blk + 1 < NBLK:
                pltpu.make_async_copy(src_hbm.at[wid, pl.ds(0, BLK)],
                                      src_t.at[1 - p], sems[1 - p]).wait()
                wait_s(0)
                start_g(1 - p, 0, 0)
                wait_s(1)
                start_g(1 - p, 1, 1)
            else:
                wait_s(0)
                wait_s(1)

        plsc.subcore_barrier()
        pltpu.sync_copy(acc.at[pl.ds(s * ROWS_PT, ROWS_PT)],
                        out_hbm.at[c].at[pl.ds(s * ROWS_PT, ROWS_PT)])

    return k(x, src3, dst3, zeros_rows)


def _sc_degrees(src3, dst3, zeros_1d, ones_1d):
    @functools.partial(
        pl.kernel,
        mesh=_sc_mesh(),
        out_type=(jax.ShapeDtypeStruct((NCORES, NPAD1), jnp.float32),
                  jax.ShapeDtypeStruct((NCORES, NPAD1), jnp.float32)),
        scratch_types=[
            pltpu.VMEM((CPT, CHUNK), jnp.int32),
            pltpu.VMEM((CPT, CHUNK), jnp.int32),
            pltpu.VMEM((CHUNK,), jnp.float32),
            pltpu.VMEM_SHARED((NPAD1,), jnp.float32),
            pltpu.VMEM_SHARED((NPAD1,), jnp.float32),
            pltpu.SemaphoreType.DMA,
            pltpu.SemaphoreType.DMA,
        ],
    )
    def k(src_hbm, dst_hbm, zeros_hbm, ones_hbm, din_hbm, dout_hbm,
          src_t, dst_t, ones_t, acc_in, acc_out, semi, semo):
        c = lax.axis_index("c")
        s = lax.axis_index("s")
        wid = c * NSUB + s
        pltpu.sync_copy(src_hbm.at[wid], src_t)
        pltpu.sync_copy(dst_hbm.at[wid], dst_t)
        pltpu.sync_copy(ones_hbm, ones_t)
        sl = pl.ds(s * ROWS1, ROWS1)
        pltpu.sync_copy(zeros_hbm, acc_in.at[sl])
        pltpu.sync_copy(zeros_hbm, acc_out.at[sl])
        plsc.subcore_barrier()

        def fire(j, carry):
            pltpu.async_copy(ones_t, acc_in.at[dst_t.at[j]], semi, add=True)
            pltpu.async_copy(ones_t, acc_out.at[src_t.at[j]], semo, add=True)
            return carry

        lax.fori_loop(0, CPT, fire, 0)

        def drain(j, carry):
            pltpu.make_async_copy(ones_t, acc_in.at[dst_t.at[0]],
                                  semi).wait()
            pltpu.make_async_copy(ones_t, acc_out.at[src_t.at[0]],
                                  semo).wait()
            return carry

        lax.fori_loop(0, CPT, drain, 0)

        plsc.subcore_barrier()
        pltpu.sync_copy(acc_in.at[sl], din_hbm.at[c].at[sl])
        pltpu.sync_copy(acc_out.at[sl], dout_hbm.at[c].at[sl])

    return k(src3, dst3, zeros_1d, ones_1d)


def _norms(dia, dib, doa, dob):
    inorm = lax.rsqrt(jnp.maximum(dia[...] + dib[...], 1.0)[:, :1])
    onorm = lax.rsqrt(jnp.maximum(doa[...] + dob[...], 1.0)[:, :1])
    return inorm, onorm


_row_spec = pl.BlockSpec((BN, D), lambda i: (i, 0))
_deg_spec = pl.BlockSpec((BN, 1), lambda i: (i, 0))
_w_spec = pl.BlockSpec((D, D), lambda i: (0, 0))
_b_spec = pl.BlockSpec((1, D), lambda i: (0, 0))


def _tc_prep(f, dia, dib, doa, dob):
    def body(f_ref, dia_r, dib_r, doa_r, dob_r, g_ref, r_ref):
        inorm, onorm = _norms(dia_r, dib_r, doa_r, dob_r)
        x = f_ref[...]
        g_ref[...] = x * onorm
        r_ref[...] = x * inorm

    return pl.pallas_call(
        body,
        grid=(N // BN,),
        in_specs=[_row_spec, _deg_spec, _deg_spec, _deg_spec, _deg_spec],
        out_specs=(_row_spec, _row_spec),
        out_shape=(jax.ShapeDtypeStruct((N, D), jnp.float32),
                   jax.ShapeDtypeStruct((N, D), jnp.float32)),
    )(f, dia, dib, doa, dob)


_p0_spec = pl.BlockSpec((1, BN, D), lambda i: (0, i, 0))
_p1_spec = pl.BlockSpec((1, BN, D), lambda i: (1, i, 0))


def _tc_layer(P, dia, dib, doa, dob, W, b, Q):
    has_res = Q is not None

    def body(p0_r, p1_r, dia_r, dib_r, doa_r, dob_r, w_r, b_r, *rest):
        g_ref = rest[-1]
        inorm, onorm = _norms(dia_r, dib_r, doa_r, dob_r)
        agg = jnp.dot(p0_r[0] + p1_r[0], w_r[...],
                      preferred_element_type=jnp.float32)
        h = agg * inorm + b_r[...]
        if has_res:
            h = h + (rest[0][0] + rest[1][0]) * inorm
        h = jnp.maximum(h, 0.0)
        g_ref[...] = h * onorm

    in_specs = [_p0_spec, _p1_spec, _deg_spec, _deg_spec, _deg_spec,
                _deg_spec, _w_spec, _b_spec]
    args = [P, P, dia, dib, doa, dob, W, b.reshape(1, D)]
    if has_res:
        in_specs += [_p0_spec, _p1_spec]
        args += [Q, Q]

    return pl.pallas_call(
        body,
        grid=(N // BN,),
        in_specs=in_specs,
        out_specs=_row_spec,
        out_shape=jax.ShapeDtypeStruct((N, D), jnp.float32),
    )(*args)


def _tc_final(P, dia, dib, W, b):
    def body(p0_r, p1_r, dia_r, dib_r, w_r, b_r, o_ref):
        inorm = lax.rsqrt(jnp.maximum(dia_r[...] + dib_r[...], 1.0)[:, :1])
        agg = jnp.dot(p0_r[0] + p1_r[0], w_r[...],
                      preferred_element_type=jnp.float32)
        o_ref[...] = agg * inorm + b_r[...]

    return pl.pallas_call(
        body,
        grid=(N // BN,),
        in_specs=[_p0_spec, _p1_spec, _deg_spec, _deg_spec, _w_spec,
                  _b_spec],
        out_specs=_row_spec,
        out_shape=jax.ShapeDtypeStruct((N, D), jnp.float32),
    )(P, P, dia, dib, W, b.reshape(1, D))


def kernel(features, edge_index, W0, b0, W1, b1, W2, b2, W3, b3):
    src = edge_index[0].astype(jnp.int32)
    dst = edge_index[1].astype(jnp.int32)
    pad = EPAD - E

    def _pad3(v, fill):
        return jnp.concatenate([v, jnp.full((pad,), fill, jnp.int32)]
                               ).reshape(NTILES, CPT, CHUNK)

    src3 = _pad3(src, 0)
    dst3 = _pad3(dst, DUMMY)
    src3_s = _pad3(src, DUMMY)
    zeros_rows = jnp.zeros((ROWS_PT, D), jnp.float32)
    zeros_1d = jnp.zeros((ROWS1,), jnp.float32)
    ones_1d = jnp.ones((CHUNK,), jnp.float32)

    din_p, dout_p = _sc_degrees(src3_s, dst3, zeros_1d, ones_1d)
    dia, dib = din_p[0, :N, None], din_p[1, :N, None]
    doa, dob = dout_p[0, :N, None], dout_p[1, :N, None]

    g, r = _tc_prep(features, dia, dib, doa, dob)
    P = _sc_spmv(g, src3, dst3, zeros_rows)
    Q = _sc_spmv(r, src3, dst3, zeros_rows)

    g1 = _tc_layer(P, dia, dib, doa, dob, W0, b0, None)
    R1 = _sc_spmv(g1, src3, dst3, zeros_rows)
    g2 = _tc_layer(R1, dia, dib, doa, dob, W1, b1, Q)
    R2 = _sc_spmv(g2, src3, dst3, zeros_rows)
    g3 = _tc_layer(R2, dia, dib, doa, dob, W2, b2, Q)
    R3 = _sc_spmv(g3, src3, dst3, zeros_rows)
    return _tc_final(R3, dia, dib, W3, b3)
